# baseline (speedup 1.0000x reference)
import functools
import math

import jax
import jax.numpy as jnp
from jax import lax
from jax.experimental import pallas as pl
from jax.experimental.pallas import tpu as pltpu
from jax.experimental.pallas import tpu_sc as plsc

D_MODEL = 64
SCALE = math.sqrt(D_MODEL)
LANES = 16

_INFO = plsc.get_sparse_core_info()
_NC = _INFO.num_cores
_NS = _INFO.num_subcores
_NW = _NC * _NS

IDX_W = 128
K = 4
CHUNK = K * IDX_W


@functools.lru_cache(maxsize=None)
def _build_gather(n_idx_rows: int, vocab: int):
    assert n_idx_rows % _NW == 0
    rows_per_w = n_idx_rows // _NW
    n_chunks = rows_per_w // K
    assert n_chunks * K == rows_per_w and n_chunks % 2 == 0
    half = n_chunks // 2

    mesh = plsc.VectorSubcoreMesh(core_axis_name="c", subcore_axis_name="s")

    @functools.partial(
        pl.kernel,
        mesh=mesh,
        out_type=jax.ShapeDtypeStruct((n_idx_rows * IDX_W, D_MODEL), jnp.float32),
        scratch_types=[
            pltpu.VMEM((K, IDX_W), jnp.int32),
            pltpu.VMEM((K, IDX_W), jnp.int32),
            pltpu.VMEM((CHUNK, D_MODEL), jnp.float32),
            pltpu.VMEM((CHUNK, D_MODEL), jnp.float32),
            pltpu.SemaphoreType.DMA,
            pltpu.SemaphoreType.DMA,
        ],
        compiler_params=pltpu.CompilerParams(use_tc_tiling_on_sc=False),
    )
    def gather_kernel(idx_hbm, table_hbm, out_hbm,
                      idx_a, idx_b, rows_a, rows_b, sem_a, sem_b):
        wid = lax.axis_index("s") * _NC + lax.axis_index("c")
        row0 = wid * rows_per_w

        def load_idx(g, idx_v):
            pltpu.sync_copy(idx_hbm.at[pl.ds(row0 + g * K, K)], idx_v)

        def fire(idx_v, rows_v, sem):
            for j in range(K):
                pltpu.async_copy(
                    table_hbm.at[idx_v.at[j]],
                    rows_v.at[pl.ds(j * IDX_W, IDX_W)],
                    sem,
                )

        def wait_chunk(rows_v, sem):
            pltpu.make_async_copy(
                table_hbm.at[pl.ds(0, CHUNK)], rows_v, sem).wait()

        def scale_store(g, rows_v):
            def body(r, carry):
                for c in range(D_MODEL // LANES):
                    v = rows_v[r, pl.ds(c * LANES, LANES)]
                    rows_v[r, pl.ds(c * LANES, LANES)] = v * SCALE
                return carry
            lax.fori_loop(0, CHUNK, body, 0, unroll=2)
            out_base = (row0 + g * K) * IDX_W
            pltpu.sync_copy(rows_v, out_hbm.at[pl.ds(out_base, CHUNK)])

        load_idx(0, idx_a)
        fire(idx_a, rows_a, sem_a)

        def loop_body(t, carry):
            ga = 2 * t
            gb = 2 * t + 1
            load_idx(gb, idx_b)
            fire(idx_b, rows_b, sem_b)
            wait_chunk(rows_a, sem_a)
            scale_store(ga, rows_a)

            @pl.when(t < half - 1)
            def _():
                load_idx(ga + 2, idx_a)
                fire(idx_a, rows_a, sem_a)

            wait_chunk(rows_b, sem_b)
            scale_store(gb, rows_b)
            return carry

        lax.fori_loop(0, half, loop_body, 0)

    return gather_kernel


def kernel(x, table):
    b, s = x.shape
    vocab, d = table.shape
    assert d == D_MODEL
    total = b * s
    assert total % IDX_W == 0
    idx = x.reshape(total // IDX_W, IDX_W).astype(jnp.int32)
    out = _build_gather(total // IDX_W, vocab)(idx, table)
    return out.reshape(b, s, D_MODEL)
